# baseline (speedup 1.0000x reference)
import functools

import jax
import jax.numpy as jnp
from jax import lax
from jax.experimental import pallas as pl
from jax.experimental.pallas import tpu as pltpu
from jax.experimental.pallas import tpu_sc as plsc

EMBED = 64
NC = 2
NS = 16
NW = NC * NS


@functools.lru_cache(maxsize=None)
def _make_gather(B, C):
    b_per_w = B // NW
    nchunks = b_per_w // C
    mesh = plsc.VectorSubcoreMesh(core_axis_name="c", subcore_axis_name="s")

    @functools.partial(
        pl.kernel,
        mesh=mesh,
        out_type=jax.ShapeDtypeStruct((B, EMBED), jnp.float32),
        scratch_types=[
            pltpu.VMEM((b_per_w,), jnp.int32),
            pltpu.VMEM((C, EMBED), jnp.float32),
            pltpu.SemaphoreType.DMA,
        ],
        compiler_params=pltpu.CompilerParams(use_tc_tiling_on_sc=False),
    )
    def k(seq_hbm, table_hbm, out_hbm, idx_v, rows_v, sem):
        wid = lax.axis_index("s") * NC + lax.axis_index("c")
        base = wid * b_per_w
        pltpu.sync_copy(seq_hbm.at[pl.ds(base, b_per_w)], idx_v)

        def body(c, carry):
            off = c * C
            pltpu.async_copy(
                table_hbm.at[idx_v.at[pl.ds(off, C)]], rows_v, sem
            ).wait()
            pltpu.sync_copy(rows_v, out_hbm.at[pl.ds(base + off, C)])
            return carry

        lax.fori_loop(0, nchunks, body, 0)

    return k


def kernel(sequence, table):
    Bdim, Ldim = sequence.shape
    B = Bdim * Ldim
    seq_flat = sequence.reshape(B)
    out = _make_gather(B, 512)(seq_flat, table)
    return out.reshape(Bdim, Ldim, EMBED)


# speedup vs baseline: 1.0252x; 1.0252x over previous
import functools

import jax
import jax.numpy as jnp
from jax import lax
from jax.experimental import pallas as pl
from jax.experimental.pallas import tpu as pltpu
from jax.experimental.pallas import tpu_sc as plsc

EMBED = 64
NC = 2
NS = 16
NW = NC * NS

NBUF = 4
AHEAD = 2


@functools.lru_cache(maxsize=None)
def _make_gather(B, C):
    b_per_w = B // NW
    nchunks = b_per_w // C
    assert b_per_w % C == 0
    assert (nchunks - 2 * AHEAD) % NBUF == 0 and nchunks >= 2 * AHEAD + NBUF
    mesh = plsc.VectorSubcoreMesh(core_axis_name="c", subcore_axis_name="s")

    @functools.partial(
        pl.kernel,
        mesh=mesh,
        out_type=jax.ShapeDtypeStruct((B, EMBED), jnp.float32),
        scratch_types=[
            pltpu.VMEM((b_per_w,), jnp.int32),
            pltpu.VMEM((NBUF, C, EMBED), jnp.float32),
            pltpu.SemaphoreType.DMA((NBUF,)),
            pltpu.SemaphoreType.DMA((NBUF,)),
        ],
        compiler_params=pltpu.CompilerParams(use_tc_tiling_on_sc=False),
    )
    def k(seq_hbm, table_hbm, out_hbm, idx_v, bufs, gsem, wsem):
        wid = lax.axis_index("s") * NC + lax.axis_index("c")
        base = wid * b_per_w
        pltpu.sync_copy(seq_hbm.at[pl.ds(base, b_per_w)], idx_v)

        def fire_gather(c, b):
            pltpu.async_copy(
                table_hbm.at[idx_v.at[pl.ds(c * C, C)]], bufs.at[b], gsem.at[b]
            )

        def wait_gather(b):
            pltpu.make_async_copy(
                table_hbm.at[idx_v.at[pl.ds(0, C)]], bufs.at[b], gsem.at[b]
            ).wait()

        def fire_write(c, b):
            pltpu.async_copy(
                bufs.at[b], out_hbm.at[pl.ds(base + c * C, C)], wsem.at[b]
            )

        def wait_write(b):
            pltpu.make_async_copy(
                bufs.at[b], out_hbm.at[pl.ds(base, C)], wsem.at[b]
            ).wait()

        for c in range(AHEAD):
            fire_gather(c, c % NBUF)
        for c in range(AHEAD):
            b = c % NBUF
            wait_gather(b)
            fire_write(c, b)
            fire_gather(c + AHEAD, (c + AHEAD) % NBUF)

        ngroups = (nchunks - 2 * AHEAD) // NBUF

        def body(g, carry):
            c0 = AHEAD + NBUF * g
            for j in range(NBUF):
                c = c0 + j
                b = (AHEAD + j) % NBUF
                wait_gather(b)
                fire_write(c, b)
                b2 = (AHEAD + j + AHEAD) % NBUF
                wait_write(b2)
                fire_gather(c + AHEAD, b2)
            return carry

        lax.fori_loop(0, ngroups, body, 0)

        for c in range(nchunks - AHEAD, nchunks):
            b = c % NBUF
            wait_gather(b)
            fire_write(c, b)
        for c in range(nchunks - NBUF, nchunks):
            wait_write(c % NBUF)

    return k


def kernel(sequence, table):
    Bdim, Ldim = sequence.shape
    B = Bdim * Ldim
    seq_flat = sequence.reshape(B)
    out = _make_gather(B, 320)(seq_flat, table)
    return out.reshape(Bdim, Ldim, EMBED)


# speedup vs baseline: 1.0508x; 1.0250x over previous
import functools

import jax
import jax.numpy as jnp
from jax import lax
from jax.experimental import pallas as pl
from jax.experimental.pallas import tpu as pltpu
from jax.experimental.pallas import tpu_sc as plsc

EMBED = 64
NC = 2
NS = 16
NW = NC * NS

NBUF = 4
AHEAD = 2


@functools.lru_cache(maxsize=None)
def _make_gather(B, C):
    b_per_w = B // NW
    nchunks = b_per_w // C
    assert b_per_w % C == 0
    assert (nchunks - 2 * AHEAD) % NBUF == 0 and nchunks >= 2 * AHEAD + NBUF
    mesh = plsc.VectorSubcoreMesh(core_axis_name="c", subcore_axis_name="s")

    @functools.partial(
        pl.kernel,
        mesh=mesh,
        out_type=jax.ShapeDtypeStruct((B, EMBED), jnp.float32),
        scratch_types=[
            pltpu.VMEM((b_per_w,), jnp.int32),
            pltpu.VMEM((NBUF, C, EMBED), jnp.float32),
            pltpu.SemaphoreType.DMA((NBUF,)),
            pltpu.SemaphoreType.DMA((NBUF,)),
        ],
        compiler_params=pltpu.CompilerParams(use_tc_tiling_on_sc=False),
    )
    def k(seq_hbm, table_hbm, out_hbm, idx_v, bufs, gsem, wsem):
        wid = lax.axis_index("s") * NC + lax.axis_index("c")
        base = wid * b_per_w
        pltpu.sync_copy(seq_hbm.at[pl.ds(base, b_per_w)], idx_v)

        def fire_gather(c, b):
            pltpu.async_copy(
                table_hbm.at[idx_v.at[pl.ds(c * C, C)]], bufs.at[b], gsem.at[b]
            )

        def wait_gather(b):
            pltpu.make_async_copy(
                table_hbm.at[idx_v.at[pl.ds(0, C)]], bufs.at[b], gsem.at[b]
            ).wait()

        def fire_write(c, b):
            pltpu.async_copy(
                bufs.at[b], out_hbm.at[pl.ds(base + c * C, C)], wsem.at[b]
            )

        def wait_write(b):
            pltpu.make_async_copy(
                bufs.at[b], out_hbm.at[pl.ds(base, C)], wsem.at[b]
            ).wait()

        for c in range(AHEAD):
            fire_gather(c, c % NBUF)
        for c in range(AHEAD):
            b = c % NBUF
            wait_gather(b)
            fire_write(c, b)
            fire_gather(c + AHEAD, (c + AHEAD) % NBUF)

        ngroups = (nchunks - 2 * AHEAD) // NBUF

        def body(g, carry):
            c0 = AHEAD + NBUF * g
            for j in range(NBUF):
                c = c0 + j
                b = (AHEAD + j) % NBUF
                wait_gather(b)
                fire_write(c, b)
                b2 = (AHEAD + j + AHEAD) % NBUF
                wait_write(b2)
                fire_gather(c + AHEAD, b2)
            return carry

        lax.fori_loop(0, ngroups, body, 0)

        for c in range(nchunks - AHEAD, nchunks):
            b = c % NBUF
            wait_gather(b)
            fire_write(c, b)
        for c in range(nchunks - NBUF, nchunks):
            wait_write(c % NBUF)

    return k


def kernel(sequence, table):
    Bdim, Ldim = sequence.shape
    B = Bdim * Ldim
    seq_lm = sequence.T.reshape(B)
    out = _make_gather(B, 320)(seq_lm, table)
    return out.reshape(Ldim, Bdim, EMBED).transpose(1, 0, 2)
